# baseline (speedup 1.0000x reference)
import functools

import jax
import jax.numpy as jnp
from jax import lax
from jax.experimental import pallas as pl
from jax.experimental.pallas import tpu as pltpu
from jax.experimental.pallas import tpu_sc as plsc

_D = 64
_SCALE = 8.0
_NW = 32
_LANES = 16
_C0 = 128
_NI = 4
_NB = 2


def _emb_lookup(tok, table):
    bsz, seq = tok.shape
    rows_per_w = bsz // _NW
    c1 = seq - _C0
    mesh = plsc.VectorSubcoreMesh(core_axis_name="c", subcore_axis_name="s")

    @functools.partial(
        pl.kernel,
        mesh=mesh,
        out_type=jax.ShapeDtypeStruct((bsz, seq, _D), jnp.float32),
        compiler_params=pltpu.CompilerParams(use_tc_tiling_on_sc=False),
        scratch_types=(
            [pltpu.VMEM((_C0,), jnp.int32) for _ in range(_NI)]
            + [pltpu.VMEM((c1,), jnp.int32) for _ in range(_NI)]
            + [pltpu.VMEM((seq, _D), jnp.float32) for _ in range(2 * _NB)]
            + [pltpu.SemaphoreType.DMA for _ in range(_NI + 2 * _NB)]
        ),
    )
    def body(tok_hbm, table_hbm, out_hbm, *refs):
        idxas = refs[:_NI]
        idxbs = refs[_NI:2 * _NI]
        gbufs = refs[2 * _NI:2 * _NI + _NB]
        sbufs = refs[2 * _NI + _NB:2 * _NI + 2 * _NB]
        isems = refs[2 * _NI + 2 * _NB:3 * _NI + 2 * _NB]
        gsems = refs[3 * _NI + 2 * _NB:3 * _NI + 3 * _NB]
        ssems = refs[3 * _NI + 3 * _NB:]

        cid = lax.axis_index("c")
        sid = lax.axis_index("s")
        wid = sid * 2 + cid
        row0 = wid * rows_per_w

        def idx_copies(j, i):
            row = row0 + j
            return (
                pltpu.make_async_copy(
                    tok_hbm.at[row, pl.ds(0, _C0)], idxas[i], isems[i]
                ),
                pltpu.make_async_copy(
                    tok_hbm.at[row, pl.ds(_C0, c1)], idxbs[i], isems[i]
                ),
            )

        def stage_idx(j, i):
            for cp in idx_copies(j, i):
                cp.start()

        def gather_copies(i, g):
            return (
                pltpu.make_async_copy(
                    table_hbm.at[idxas[i]],
                    gbufs[g].at[pl.ds(0, _C0)],
                    gsems[g],
                ),
                pltpu.make_async_copy(
                    table_hbm.at[idxbs[i]],
                    gbufs[g].at[pl.ds(_C0, c1)],
                    gsems[g],
                ),
            )

        def start_gathers(j, i, g):
            for cp in idx_copies(j, i):
                cp.wait()
            for cp in gather_copies(i, g):
                cp.start()

        def wait_gathers(i, g):
            for cp in gather_copies(i, g):
                cp.wait()

        def start_store(j, s):
            pltpu.make_async_copy(
                sbufs[s], out_hbm.at[row0 + j], ssems[s]
            ).start()

        def wait_store(j, s):
            pltpu.make_async_copy(
                sbufs[s], out_hbm.at[row0 + j], ssems[s]
            ).wait()

        def scale(g, s):
            gbuf, sbuf = gbufs[g], sbufs[s]

            def sbody(r, carry):
                for col in range(0, _D, _LANES):
                    sbuf[r, pl.ds(col, _LANES)] = (
                        gbuf[r, pl.ds(col, _LANES)] * _SCALE
                    )
                return carry

            lax.fori_loop(0, seq, sbody, 0, unroll=8)

        for j in range(_NI):
            stage_idx(j, j)
        start_gathers(0, 0, 0)
        start_gathers(1, 1, 1)

        for j in range(_NI):
            g = j % _NB
            wait_gathers(j % _NI, g)
            stage_idx(j + _NI, j % _NI)
            if j >= _NB:
                wait_store(j - _NB, g)
            scale(g, g)
            start_store(j, g)
            start_gathers(j + _NB, (j + _NB) % _NI, g)

        def loop_body(ii, carry):
            for b in range(_NI):
                j = ii * _NI + b
                g = b % _NB
                wait_gathers(b, g)

                @pl.when(j + _NI < rows_per_w)
                def _():
                    stage_idx(j + _NI, b)

                wait_store(j - _NB, g)
                scale(g, g)
                start_store(j, g)

                @pl.when(j + _NB < rows_per_w)
                def _():
                    start_gathers(j + _NB, (b + _NB) % _NI, g)
            return carry

        lax.fori_loop(1, rows_per_w // _NI, loop_body, 0)

        wait_store(rows_per_w - 2, 0)
        wait_store(rows_per_w - 1, 1)

    return body(tok, table)


def kernel(tokens, embedding):
    tok = tokens if tokens.dtype == jnp.int32 else tokens.astype(jnp.int32)
    return _emb_lookup(tok, embedding)


# speedup vs baseline: 1.0100x; 1.0100x over previous
import functools

import jax
import jax.numpy as jnp
from jax import lax
from jax.experimental import pallas as pl
from jax.experimental.pallas import tpu as pltpu
from jax.experimental.pallas import tpu_sc as plsc

_D = 64
_SCALE = 8.0
_NW = 32
_C0 = 128
_NR = 4


def _emb_lookup(tok, table):
    bsz, seq = tok.shape
    rows_per_w = bsz // _NW
    c1 = seq - _C0
    mesh = plsc.VectorSubcoreMesh(core_axis_name="c", subcore_axis_name="s")

    @functools.partial(
        pl.kernel,
        mesh=mesh,
        out_type=jax.ShapeDtypeStruct((bsz, seq, _D), jnp.float32),
        compiler_params=pltpu.CompilerParams(use_tc_tiling_on_sc=False),
        scratch_types=(
            [pltpu.VMEM((_C0,), jnp.int32) for _ in range(_NR)]
            + [pltpu.VMEM((c1,), jnp.int32) for _ in range(_NR)]
            + [pltpu.VMEM((seq, _D), jnp.float32) for _ in range(_NR)]
            + [pltpu.SemaphoreType.DMA for _ in range(3 * _NR)]
        ),
    )
    def body(tok_hbm, table_hbm, out_hbm, *refs):
        idxas = refs[:_NR]
        idxbs = refs[_NR:2 * _NR]
        bufs = refs[2 * _NR:3 * _NR]
        isems = refs[3 * _NR:4 * _NR]
        gsems = refs[4 * _NR:5 * _NR]
        ssems = refs[5 * _NR:]

        cid = lax.axis_index("c")
        sid = lax.axis_index("s")
        wid = sid * 2 + cid
        row0 = wid * rows_per_w

        def idx_copies(j, i):
            row = row0 + j
            return (
                pltpu.make_async_copy(
                    tok_hbm.at[row, pl.ds(0, _C0)], idxas[i], isems[i]
                ),
                pltpu.make_async_copy(
                    tok_hbm.at[row, pl.ds(_C0, c1)], idxbs[i], isems[i]
                ),
            )

        def stage_idx(j, i):
            for cp in idx_copies(j, i):
                cp.start()

        def gather_copies(g):
            return (
                pltpu.make_async_copy(
                    table_hbm.at[idxas[g]],
                    bufs[g].at[pl.ds(0, _C0)],
                    gsems[g],
                ),
                pltpu.make_async_copy(
                    table_hbm.at[idxbs[g]],
                    bufs[g].at[pl.ds(_C0, c1)],
                    gsems[g],
                ),
            )

        def start_gathers(j, g):
            for cp in idx_copies(j, g):
                cp.wait()
            for cp in gather_copies(g):
                cp.start()

        def wait_gathers(g):
            for cp in gather_copies(g):
                cp.wait()

        def start_store(j, s):
            pltpu.make_async_copy(
                bufs[s], out_hbm.at[row0 + j], ssems[s]
            ).start()

        def wait_store(j, s):
            pltpu.make_async_copy(
                bufs[s], out_hbm.at[row0 + j], ssems[s]
            ).wait()

        for j in range(_NR):
            stage_idx(j, j)
        start_gathers(0, 0)
        start_gathers(1, 1)

        for j in range(_NR):
            wait_gathers(j)
            stage_idx(j + _NR, j)
            start_store(j, j)
            nxt = j + 2
            if nxt < _NR:
                start_gathers(nxt, nxt)
            else:
                wait_store(nxt - _NR, nxt % _NR)
                start_gathers(nxt, nxt % _NR)

        def loop_body(ii, carry):
            for b in range(_NR):
                j = ii * _NR + b
                wait_gathers(b)

                @pl.when(j + _NR < rows_per_w)
                def _():
                    stage_idx(j + _NR, b)

                start_store(j, b)

                @pl.when(j + 2 < rows_per_w)
                def _():
                    g = (b + 2) % _NR
                    wait_store(j - 2, g)
                    start_gathers(j + 2, g)
            return carry

        lax.fori_loop(1, rows_per_w // _NR, loop_body, 0)

        for k in range(_NR):
            j = rows_per_w - _NR + k
            wait_store(j, j % _NR)

    return body(tok, table)


def kernel(tokens, embedding):
    tok = tokens if tokens.dtype == jnp.int32 else tokens.astype(jnp.int32)
    return _emb_lookup(tok, embedding * _SCALE)


# speedup vs baseline: 1.0470x; 1.0366x over previous
import functools

import jax
import jax.numpy as jnp
from jax import lax
from jax.experimental import pallas as pl
from jax.experimental.pallas import tpu as pltpu
from jax.experimental.pallas import tpu_sc as plsc

_D = 64
_SCALE = 8.0
_NW = 32
_C0 = 128
_NR = 4


def _emb_lookup(tok, table):
    bsz, seq = tok.shape
    rows_per_w = bsz // _NW
    c1 = seq - _C0
    mesh = plsc.VectorSubcoreMesh(core_axis_name="c", subcore_axis_name="s")

    @functools.partial(
        pl.kernel,
        mesh=mesh,
        out_type=jax.ShapeDtypeStruct((bsz, seq, _D), jnp.float32),
        compiler_params=pltpu.CompilerParams(use_tc_tiling_on_sc=False),
        scratch_types=(
            [pltpu.VMEM((_C0,), jnp.int32) for _ in range(_NR)]
            + [pltpu.VMEM((c1,), jnp.int32) for _ in range(_NR)]
            + [pltpu.VMEM((seq, _D), jnp.float32) for _ in range(_NR)]
            + [pltpu.SemaphoreType.DMA for _ in range(3 * _NR)]
        ),
    )
    def body(tok_hbm, table_hbm, out_hbm, *refs):
        idxas = refs[:_NR]
        idxbs = refs[_NR:2 * _NR]
        bufs = refs[2 * _NR:3 * _NR]
        isems = refs[3 * _NR:4 * _NR]
        gsems = refs[4 * _NR:5 * _NR]
        ssems = refs[5 * _NR:]

        cid = lax.axis_index("c")
        sid = lax.axis_index("s")
        wid = sid * 2 + cid
        row0 = wid * rows_per_w

        def idx_copies(j, i):
            row = row0 + j
            return (
                pltpu.make_async_copy(
                    tok_hbm.at[row, pl.ds(0, _C0)], idxas[i], isems[i]
                ),
                pltpu.make_async_copy(
                    tok_hbm.at[row, pl.ds(_C0, c1)], idxbs[i], isems[i]
                ),
            )

        def stage_idx(j, i):
            for cp in idx_copies(j, i):
                cp.start()

        def gather_copies(g):
            return (
                pltpu.make_async_copy(
                    table_hbm.at[idxas[g]],
                    bufs[g].at[pl.ds(0, _C0)],
                    gsems[g],
                ),
                pltpu.make_async_copy(
                    table_hbm.at[idxbs[g]],
                    bufs[g].at[pl.ds(_C0, c1)],
                    gsems[g],
                ),
            )

        def start_gathers(j, g):
            for cp in idx_copies(j, g):
                cp.wait()
            for cp in gather_copies(g):
                cp.start()

        def wait_gathers(g):
            for cp in gather_copies(g):
                cp.wait()

        def start_store(j, s):
            pltpu.make_async_copy(
                bufs[s], out_hbm.at[row0 + j], ssems[s]
            ).start()

        def wait_store(j, s):
            pltpu.make_async_copy(
                bufs[s], out_hbm.at[row0 + j], ssems[s]
            ).wait()

        for j in range(_NR):
            stage_idx(j, j)
        start_gathers(0, 0)
        start_gathers(1, 1)

        for j in range(_NR):
            wait_gathers(j)
            stage_idx(j + _NR, j)
            start_store(j, j)
            nxt = j + 2
            if nxt < _NR:
                start_gathers(nxt, nxt)
            else:
                wait_store(nxt - _NR, nxt % _NR)
                start_gathers(nxt, nxt % _NR)

        def loop_body(ii, carry):
            for b in range(_NR):
                j = ii * _NR + b
                wait_gathers(b)

                @pl.when(j + _NR < rows_per_w)
                def _():
                    stage_idx(j + _NR, b)

                start_store(j, b)

                @pl.when(j + 2 < rows_per_w)
                def _():
                    g = (b + 2) % _NR
                    wait_store(j - 2, g)
                    start_gathers(j + 2, g)
            return carry

        lax.fori_loop(1, rows_per_w // _NR, loop_body, 0)

        for k in range(_NR):
            j = rows_per_w - _NR + k
            wait_store(j, j % _NR)

    return body(tok, table)


def kernel(tokens, embedding):
    tok = tokens if tokens.dtype == jnp.int32 else tokens.astype(jnp.int32)
    return _emb_lookup(tok, embedding) * _SCALE
